# Initial kernel scaffold; baseline (speedup 1.0000x reference)
import jax
import jax.numpy as jnp
from jax.experimental import pallas as pl


def kernel(x, conv_w0, conv_b0, conv_w1, conv_b1, conv_w2, conv_b2, conv_w3, conv_b3, conv_w4, conv_b4, sel0, sel1, sel2, sel3, lin_w0, lin_b0, lin_w1, lin_b1, lin_w2, lin_b2, lin_w3, lin_b3):
    raise NotImplementedError("write your pallas kernel here")


# speedup vs baseline: 1.4107x; 1.4107x over previous
import jax
import jax.numpy as jnp
from jax.experimental import pallas as pl
from jax.experimental.pallas import tpu as pltpu

_H0, _W0 = 70, 320
_B_CONV = [(3, 24, 5, 2, 35, 160, 33, 158),
           (24, 36, 5, 2, 17, 79, 15, 77),
           (36, 48, 5, 2, 8, 39, 6, 37),
           (48, 64, 3, 1, 6, 37, 4, 35),
           (64, 64, 3, 1, 4, 35, 2, 33)]
_KHG5 = (0, 2, 4, 1, 3)


def _elu(v):
    return jnp.where(v > 0.0, v, jnp.exp(v) - 1.0)


def _conv_body(xee, xeo, xoe, xoo,
               w1, b1, w2, b2, w3, b3, w4, b4, w5, b5,
               selA, selB,
               o_ref,
               slab1, slab2, slab3, slab4, slab5,
               s1ee, s1eo, s1oe, s1oo, s2ee, s2eo, s2oe, s2oo, s3, s4):
    f32 = jnp.float32

    xe = (xee, xeo)
    xo = (xoe, xoo)
    bias1 = jnp.broadcast_to(b1[...], (24, 158))
    s1 = ((s1ee, s1eo), (s1oe, s1oo))
    for r in range(33):
        for kw in range(5):
            dw = kw // 2
            slab1[kw * 15:kw * 15 + 9, :] = xe[kw % 2][3 * r:3 * r + 9, dw:dw + 158]
            slab1[kw * 15 + 9:kw * 15 + 15, :] = xo[kw % 2][3 * r:3 * r + 6, dw:dw + 158]
        acc = _elu(jnp.dot(w1[...], slab1[...], preferred_element_type=f32) + bias1)
        par = jnp.dot(acc, selA[...], preferred_element_type=f32)
        dst = s1[r % 2]
        i = r // 2
        dst[0][24 * i:24 * i + 24, :] = par[:, :79]
        dst[1][24 * i:24 * i + 24, :] = par[:, 79:]

    bias2 = jnp.broadcast_to(b2[...], (36, 77))
    s2 = ((s2ee, s2eo), (s2oe, s2oo))
    for r in range(15):
        for kw in range(5):
            dw = kw // 2
            slab2[kw * 120:kw * 120 + 72, :] = s1[0][kw % 2][24 * r:24 * r + 72, dw:dw + 77]
            slab2[kw * 120 + 72:kw * 120 + 120, :] = s1[1][kw % 2][24 * r:24 * r + 48, dw:dw + 77]
        acc = _elu(jnp.dot(w2[...], slab2[...], preferred_element_type=f32) + bias2)
        par = jnp.dot(acc, selB[...], preferred_element_type=f32)
        dst = s2[r % 2]
        i = r // 2
        dst[0][36 * i:36 * i + 36, :] = par[:, :39]
        dst[1][36 * i:36 * i + 36, :38] = par[:, 39:]

    for r in range(6):
        for kw in range(5):
            dw = kw // 2
            slab3[kw * 180:kw * 180 + 108, 37 * r:37 * r + 37] = \
                s2[0][kw % 2][36 * r:36 * r + 108, dw:dw + 37]
            slab3[kw * 180 + 108:kw * 180 + 180, 37 * r:37 * r + 37] = \
                s2[1][kw % 2][36 * r:36 * r + 72, dw:dw + 37]
    bias3 = jnp.broadcast_to(b3[...], (48, 6 * 37))
    out3 = _elu(jnp.dot(w3[...], slab3[...], preferred_element_type=f32) + bias3)
    for r in range(6):
        s3[48 * r:48 * r + 48, :] = out3[:, 37 * r:37 * r + 37]

    for r in range(4):
        for kw in range(3):
            slab4[kw * 144:kw * 144 + 144, 35 * r:35 * r + 35] = \
                s3[48 * r:48 * r + 144, kw:kw + 35]
    bias4 = jnp.broadcast_to(b4[...], (64, 4 * 35))
    out4 = _elu(jnp.dot(w4[...], slab4[...], preferred_element_type=f32) + bias4)
    for r in range(4):
        s4[64 * r:64 * r + 64, :] = out4[:, 35 * r:35 * r + 35]

    for r in range(2):
        for kw in range(3):
            slab5[kw * 192:kw * 192 + 192, 33 * r:33 * r + 33] = \
                s4[64 * r:64 * r + 192, kw:kw + 33]
    bias5 = jnp.broadcast_to(b5[...], (64, 2 * 33))
    out5 = jnp.dot(w5[...], slab5[...], preferred_element_type=f32) + bias5
    for r in range(2):
        o_ref[64 * r:64 * r + 64, :] = out5[:, 33 * r:33 * r + 33]


def _head_body(x_ref, w1, b1, w2, b2, w3, b3, w4, b4, o_ref):
    f32 = jnp.float32
    h = x_ref[...]
    h = _elu(jnp.dot(h, w1[...], preferred_element_type=f32) + b1[...])
    h = _elu(jnp.dot(h, w2[...], preferred_element_type=f32) + b2[...])
    h = jnp.dot(h, w3[...], preferred_element_type=f32) + b3[...]
    o_ref[...] = jnp.dot(h, w4[...], preferred_element_type=f32) + b4[...]


def _cspec(a):
    zeros = (0,) * a.ndim
    return pl.BlockSpec(a.shape, lambda b, _z=zeros: _z)


def _reorder_w(wg, cin, cout, k):
    w4 = wg.reshape(k, cout, k, cin)
    if k == 5:
        w4 = w4[:, :, _KHG5, :]
    return jnp.transpose(w4, (1, 0, 2, 3)).reshape(cout, k * k * cin)


def kernel(x, conv_w0, conv_b0, conv_w1, conv_b1, conv_w2, conv_b2,
           conv_w3, conv_b3, conv_w4, conv_b4,
           sel0, sel1, sel2, sel3,
           lin_w0, lin_b0, lin_w1, lin_b1, lin_w2, lin_b2, lin_w3, lin_b3):
    B = x.shape[0]
    f32 = jnp.float32

    xt = jnp.transpose(x, (0, 2, 1, 3))
    xe, xo = xt[:, 0::2], xt[:, 1::2]
    planes = [p.reshape(B, 105, 160)
              for p in (xe[..., 0::2], xe[..., 1::2], xo[..., 0::2], xo[..., 1::2])]

    ws = [_reorder_w(w, c[0], c[1], c[2])
          for w, c in zip((conv_w0, conv_w1, conv_w2, conv_w3, conv_w4), _B_CONV)]
    bs = (conv_b0, conv_b1, conv_b2, conv_b3, conv_b4)
    selA = jnp.concatenate([sel0, sel1], axis=1)
    selB = jnp.concatenate([sel2, sel3], axis=1)

    conv_consts = []
    for w, b in zip(ws, bs):
        conv_consts += [w, b]
    conv_consts += [selA, selB]

    in_specs = [pl.BlockSpec((None, 105, 160), lambda b: (b, 0, 0))] * 4
    in_specs += [_cspec(a) for a in conv_consts]

    scratch = [
        pltpu.VMEM((75, 158), f32), pltpu.VMEM((600, 77), f32),
        pltpu.VMEM((900, 222), f32), pltpu.VMEM((432, 140), f32),
        pltpu.VMEM((576, 66), f32),
        pltpu.VMEM((408, 79), f32), pltpu.VMEM((408, 79), f32),
        pltpu.VMEM((384, 79), f32), pltpu.VMEM((384, 79), f32),
        pltpu.VMEM((288, 39), f32), pltpu.VMEM((288, 38), f32),
        pltpu.VMEM((252, 39), f32), pltpu.VMEM((252, 38), f32),
        pltpu.VMEM((288, 37), f32), pltpu.VMEM((256, 35), f32),
    ]
    feat = pl.pallas_call(
        _conv_body,
        out_shape=jax.ShapeDtypeStruct((B, 128, 33), f32),
        grid_spec=pltpu.PrefetchScalarGridSpec(
            num_scalar_prefetch=0,
            grid=(B,),
            in_specs=in_specs,
            out_specs=pl.BlockSpec((None, 128, 33), lambda b: (b, 0, 0)),
            scratch_shapes=scratch,
        ),
        compiler_params=pltpu.CompilerParams(dimension_semantics=("parallel",)),
    )(*planes, *conv_consts)

    xf = feat.reshape(B, 4224)
    lin_consts = [lin_w0, lin_b0, lin_w1, lin_b1, lin_w2, lin_b2, lin_w3, lin_b3]
    mb = B // 2 if B % 2 == 0 else B
    head = pl.pallas_call(
        _head_body,
        out_shape=jax.ShapeDtypeStruct((B, 128), f32),
        grid_spec=pltpu.PrefetchScalarGridSpec(
            num_scalar_prefetch=0,
            grid=(B // mb,),
            in_specs=[pl.BlockSpec((mb, 4224), lambda i: (i, 0))]
                     + [_cspec(a) for a in lin_consts],
            out_specs=pl.BlockSpec((mb, 128), lambda i: (i, 0)),
        ),
        compiler_params=pltpu.CompilerParams(dimension_semantics=("parallel",)),
    )(xf, *lin_consts)
    return head[:, :1]


# speedup vs baseline: 1.4757x; 1.0461x over previous
import jax
import jax.numpy as jnp
from jax.experimental import pallas as pl
from jax.experimental.pallas import tpu as pltpu

_H0, _W0 = 70, 320
_B_CONV = [(3, 24, 5, 2), (24, 36, 5, 2), (36, 48, 5, 2),
           (48, 64, 3, 1), (64, 64, 3, 1)]
_KHG5 = (0, 2, 4, 1, 3)


def _elu(v):
    return jnp.where(v > 0.0, v, jnp.exp(v) - 1.0)


def _conv_body(xp, w_ref, b_ref, sel_ref, o_ref,
               slab1, slab2, slab3, slab4, slab5,
               s1ee, s1eo, s1oe, s1oo, s2ee, s2eo, s2oe, s2oo, s3, s4):
    f32 = jnp.float32
    bf16 = jnp.bfloat16

    bias1 = jnp.broadcast_to(b_ref[0, :24, :], (24, 158))
    s1 = ((s1ee, s1eo), (s1oe, s1oo))
    w1 = w_ref[0, :24, :75]
    selA = sel_ref[:, :158]
    for r in range(33):
        for kw in range(5):
            dw = kw // 2
            slab1[kw * 15:kw * 15 + 9, :] = xp[kw % 2, 3 * r:3 * r + 9, dw:dw + 158]
            slab1[kw * 15 + 9:kw * 15 + 15, :] = xp[2 + kw % 2, 3 * r:3 * r + 6, dw:dw + 158]
        acc = _elu(jnp.dot(w1, slab1[...], preferred_element_type=f32) + bias1)
        par = jnp.dot(acc.astype(bf16), selA, preferred_element_type=f32)
        dst = s1[r % 2]
        i = r // 2
        dst[0][24 * i:24 * i + 24, :] = par[:, :79].astype(bf16)
        dst[1][24 * i:24 * i + 24, :] = par[:, 79:].astype(bf16)

    bias2 = jnp.broadcast_to(b_ref[1, :36, :], (36, 77))
    s2 = ((s2ee, s2eo), (s2oe, s2oo))
    w2 = w_ref[1, :36, :600]
    selB = sel_ref[:77, 158:235]
    for r in range(15):
        for kw in range(5):
            dw = kw // 2
            slab2[kw * 120:kw * 120 + 72, :] = s1[0][kw % 2][24 * r:24 * r + 72, dw:dw + 77]
            slab2[kw * 120 + 72:kw * 120 + 120, :] = s1[1][kw % 2][24 * r:24 * r + 48, dw:dw + 77]
        acc = _elu(jnp.dot(w2, slab2[...], preferred_element_type=f32) + bias2)
        par = jnp.dot(acc.astype(bf16), selB, preferred_element_type=f32)
        dst = s2[r % 2]
        i = r // 2
        dst[0][36 * i:36 * i + 36, :] = par[:, :39].astype(bf16)
        dst[1][36 * i:36 * i + 36, :38] = par[:, 39:].astype(bf16)

    for r in range(6):
        for kw in range(5):
            dw = kw // 2
            slab3[kw * 180:kw * 180 + 108, 37 * r:37 * r + 37] = \
                s2[0][kw % 2][36 * r:36 * r + 108, dw:dw + 37]
            slab3[kw * 180 + 108:kw * 180 + 180, 37 * r:37 * r + 37] = \
                s2[1][kw % 2][36 * r:36 * r + 72, dw:dw + 37]
    bias3 = jnp.broadcast_to(b_ref[2, :48, :], (48, 6 * 37))
    out3 = _elu(jnp.dot(w_ref[2, :48, :900], slab3[...],
                        preferred_element_type=f32) + bias3).astype(bf16)
    for r in range(6):
        s3[48 * r:48 * r + 48, :] = out3[:, 37 * r:37 * r + 37]

    for r in range(4):
        for kw in range(3):
            slab4[kw * 144:kw * 144 + 144, 35 * r:35 * r + 35] = \
                s3[48 * r:48 * r + 144, kw:kw + 35]
    bias4 = jnp.broadcast_to(b_ref[3, :64, :], (64, 4 * 35))
    out4 = _elu(jnp.dot(w_ref[3, :64, :432], slab4[...],
                        preferred_element_type=f32) + bias4).astype(bf16)
    for r in range(4):
        s4[64 * r:64 * r + 64, :] = out4[:, 35 * r:35 * r + 35]

    for r in range(2):
        for kw in range(3):
            slab5[kw * 192:kw * 192 + 192, 33 * r:33 * r + 33] = \
                s4[64 * r:64 * r + 192, kw:kw + 33]
    bias5 = jnp.broadcast_to(b_ref[4, :64, :], (64, 2 * 33))
    out5 = jnp.dot(w_ref[4, :64, :576], slab5[...],
                   preferred_element_type=f32) + bias5
    for r in range(2):
        o_ref[64 * r:64 * r + 64, :] = out5[:, 33 * r:33 * r + 33]


def _head_body(x_ref, w1, b1, w2, b2, w3, b3, w4, b4, o_ref):
    f32 = jnp.float32
    h = x_ref[...]
    h = _elu(jnp.dot(h, w1[...], preferred_element_type=f32) + b1[...])
    h = _elu(jnp.dot(h, w2[...], preferred_element_type=f32) + b2[...])
    h = jnp.dot(h, w3[...], preferred_element_type=f32) + b3[...]
    o_ref[...] = jnp.dot(h, w4[...], preferred_element_type=f32) + b4[...]


def _cspec(a):
    zeros = (0,) * a.ndim
    return pl.BlockSpec(a.shape, lambda b, _z=zeros: _z)


def _reorder_w(wg, cin, cout, k):
    w4 = wg.reshape(k, cout, k, cin)
    if k == 5:
        w4 = w4[:, :, _KHG5, :]
    return jnp.transpose(w4, (1, 0, 2, 3)).reshape(cout, k * k * cin)


def kernel(x, conv_w0, conv_b0, conv_w1, conv_b1, conv_w2, conv_b2,
           conv_w3, conv_b3, conv_w4, conv_b4,
           sel0, sel1, sel2, sel3,
           lin_w0, lin_b0, lin_w1, lin_b1, lin_w2, lin_b2, lin_w3, lin_b3):
    B = x.shape[0]
    f32 = jnp.float32
    bf16 = jnp.bfloat16

    xt = jnp.transpose(x, (0, 2, 1, 3))
    xe, xo = xt[:, 0::2], xt[:, 1::2]
    planes = jnp.stack([p.reshape(B, 105, 160)
                        for p in (xe[..., 0::2], xe[..., 1::2],
                                  xo[..., 0::2], xo[..., 1::2])],
                       axis=1).astype(bf16)

    ws = [_reorder_w(w, c[0], c[1], c[2])
          for w, c in zip((conv_w0, conv_w1, conv_w2, conv_w3, conv_w4), _B_CONV)]
    wstk = jnp.zeros((5, 64, 900), bf16)
    bstk = jnp.zeros((5, 64, 1), f32)
    for i, (w, b) in enumerate(zip(ws, (conv_b0, conv_b1, conv_b2, conv_b3, conv_b4))):
        wstk = wstk.at[i, :w.shape[0], :w.shape[1]].set(w.astype(bf16))
        bstk = bstk.at[i, :b.shape[0], :].set(b)
    sstk = jnp.zeros((158, 235), bf16)
    sstk = sstk.at[:, :158].set(jnp.concatenate([sel0, sel1], 1).astype(bf16))
    sstk = sstk.at[:77, 158:].set(jnp.concatenate([sel2, sel3], 1).astype(bf16))

    scratch = [
        pltpu.VMEM((75, 158), bf16), pltpu.VMEM((600, 77), bf16),
        pltpu.VMEM((900, 222), bf16), pltpu.VMEM((432, 140), bf16),
        pltpu.VMEM((576, 66), bf16),
        pltpu.VMEM((408, 79), bf16), pltpu.VMEM((408, 79), bf16),
        pltpu.VMEM((384, 79), bf16), pltpu.VMEM((384, 79), bf16),
        pltpu.VMEM((288, 39), bf16), pltpu.VMEM((288, 38), bf16),
        pltpu.VMEM((252, 39), bf16), pltpu.VMEM((252, 38), bf16),
        pltpu.VMEM((288, 37), bf16), pltpu.VMEM((256, 35), bf16),
    ]
    feat = pl.pallas_call(
        _conv_body,
        out_shape=jax.ShapeDtypeStruct((B, 128, 33), f32),
        grid_spec=pltpu.PrefetchScalarGridSpec(
            num_scalar_prefetch=0,
            grid=(B,),
            in_specs=[pl.BlockSpec((None, 4, 105, 160), lambda b: (b, 0, 0, 0)),
                      _cspec(wstk), _cspec(bstk), _cspec(sstk)],
            out_specs=pl.BlockSpec((None, 128, 33), lambda b: (b, 0, 0)),
            scratch_shapes=scratch,
        ),
        compiler_params=pltpu.CompilerParams(dimension_semantics=("parallel",)),
    )(planes, wstk, bstk, sstk)

    xf = feat.reshape(B, 4224)
    lin_consts = [lin_w0, lin_b0, lin_w1, lin_b1, lin_w2, lin_b2, lin_w3, lin_b3]
    mb = B // 2 if B % 2 == 0 else B
    head = pl.pallas_call(
        _head_body,
        out_shape=jax.ShapeDtypeStruct((B, 128), f32),
        grid_spec=pltpu.PrefetchScalarGridSpec(
            num_scalar_prefetch=0,
            grid=(B // mb,),
            in_specs=[pl.BlockSpec((mb, 4224), lambda i: (i, 0))]
                     + [_cspec(a) for a in lin_consts],
            out_specs=pl.BlockSpec((mb, 128), lambda i: (i, 0)),
        ),
        compiler_params=pltpu.CompilerParams(dimension_semantics=("parallel",)),
    )(xf, *lin_consts)
    return head[:, :1]


# speedup vs baseline: 1.5230x; 1.0321x over previous
import jax
import jax.numpy as jnp
from jax.experimental import pallas as pl
from jax.experimental.pallas import tpu as pltpu

_H0, _W0 = 70, 320
_B_CONV = [(3, 24, 5, 2), (24, 36, 5, 2), (36, 48, 5, 2),
           (48, 64, 3, 1), (64, 64, 3, 1)]
_KHG5 = (0, 2, 4, 1, 3)


def _elu(v):
    return jnp.where(v > 0.0, v, jnp.exp(v) - 1.0)


_IPS = 4


def _conv_body(xp, w_ref, b_ref, sel_ref, o_ref,
               slab1, slab2, slab3, slab4, slab5,
               s1ee, s1eo, s1oe, s1oo, s2ee, s2eo, s2oe, s2oo, s3, s4):
    for img in range(_IPS):
        _conv_one(xp, w_ref, b_ref, sel_ref, o_ref, img,
                  slab1, slab2, slab3, slab4, slab5,
                  s1ee, s1eo, s1oe, s1oo, s2ee, s2eo, s2oe, s2oo, s3, s4)


def _conv_one(xp4, w_ref, b_ref, sel_ref, o_ref4, img,
              slab1, slab2, slab3, slab4, slab5,
              s1ee, s1eo, s1oe, s1oo, s2ee, s2eo, s2oe, s2oo, s3, s4):
    f32 = jnp.float32
    bf16 = jnp.bfloat16

    bias1 = jnp.broadcast_to(b_ref[0, :24, :], (24, 158))
    s1 = ((s1ee, s1eo), (s1oe, s1oo))
    w1 = w_ref[0, :24, :75]
    selA = sel_ref[:, :158]
    for r in range(33):
        for kw in range(5):
            dw = kw // 2
            slab1[kw * 15:kw * 15 + 9, :] = xp4[img, kw % 2, 3 * r:3 * r + 9, dw:dw + 158]
            slab1[kw * 15 + 9:kw * 15 + 15, :] = xp4[img, 2 + kw % 2, 3 * r:3 * r + 6, dw:dw + 158]
        acc = _elu(jnp.dot(w1, slab1[...], preferred_element_type=f32) + bias1)
        par = jnp.dot(acc.astype(bf16), selA, preferred_element_type=f32)
        dst = s1[r % 2]
        i = r // 2
        dst[0][24 * i:24 * i + 24, :] = par[:, :79].astype(bf16)
        dst[1][24 * i:24 * i + 24, :] = par[:, 79:].astype(bf16)

    bias2 = jnp.broadcast_to(b_ref[1, :36, :], (36, 77))
    s2 = ((s2ee, s2eo), (s2oe, s2oo))
    w2 = w_ref[1, :36, :600]
    selB = sel_ref[:77, 158:235]
    for r in range(15):
        for kw in range(5):
            dw = kw // 2
            slab2[kw * 120:kw * 120 + 72, :] = s1[0][kw % 2][24 * r:24 * r + 72, dw:dw + 77]
            slab2[kw * 120 + 72:kw * 120 + 120, :] = s1[1][kw % 2][24 * r:24 * r + 48, dw:dw + 77]
        acc = _elu(jnp.dot(w2, slab2[...], preferred_element_type=f32) + bias2)
        par = jnp.dot(acc.astype(bf16), selB, preferred_element_type=f32)
        dst = s2[r % 2]
        i = r // 2
        dst[0][36 * i:36 * i + 36, :] = par[:, :39].astype(bf16)
        dst[1][36 * i:36 * i + 36, :38] = par[:, 39:].astype(bf16)

    for r in range(6):
        for kw in range(5):
            dw = kw // 2
            slab3[kw * 180:kw * 180 + 108, 37 * r:37 * r + 37] = \
                s2[0][kw % 2][36 * r:36 * r + 108, dw:dw + 37]
            slab3[kw * 180 + 108:kw * 180 + 180, 37 * r:37 * r + 37] = \
                s2[1][kw % 2][36 * r:36 * r + 72, dw:dw + 37]
    bias3 = jnp.broadcast_to(b_ref[2, :48, :], (48, 6 * 37))
    out3 = _elu(jnp.dot(w_ref[2, :48, :900], slab3[...],
                        preferred_element_type=f32) + bias3).astype(bf16)
    for r in range(6):
        s3[48 * r:48 * r + 48, :] = out3[:, 37 * r:37 * r + 37]

    for r in range(4):
        for kw in range(3):
            slab4[kw * 144:kw * 144 + 144, 35 * r:35 * r + 35] = \
                s3[48 * r:48 * r + 144, kw:kw + 35]
    bias4 = jnp.broadcast_to(b_ref[3, :64, :], (64, 4 * 35))
    out4 = _elu(jnp.dot(w_ref[3, :64, :432], slab4[...],
                        preferred_element_type=f32) + bias4).astype(bf16)
    for r in range(4):
        s4[64 * r:64 * r + 64, :] = out4[:, 35 * r:35 * r + 35]

    for r in range(2):
        for kw in range(3):
            slab5[kw * 192:kw * 192 + 192, 33 * r:33 * r + 33] = \
                s4[64 * r:64 * r + 192, kw:kw + 33]
    bias5 = jnp.broadcast_to(b_ref[4, :64, :], (64, 2 * 33))
    out5 = jnp.dot(w_ref[4, :64, :576], slab5[...],
                   preferred_element_type=f32) + bias5
    for r in range(2):
        o_ref4[img, 64 * r:64 * r + 64, :] = out5[:, 33 * r:33 * r + 33]


def _head_body(x_ref, w1, b1, w2, b2, w3, b3, w4, b4, o_ref):
    f32 = jnp.float32
    h = x_ref[...]
    h = _elu(jnp.dot(h, w1[...], preferred_element_type=f32) + b1[...])
    h = _elu(jnp.dot(h, w2[...], preferred_element_type=f32) + b2[...])
    h = jnp.dot(h, w3[...], preferred_element_type=f32) + b3[...]
    o_ref[...] = jnp.dot(h, w4[...], preferred_element_type=f32) + b4[...]


def _cspec(a):
    zeros = (0,) * a.ndim
    return pl.BlockSpec(a.shape, lambda b, _z=zeros: _z)


def _reorder_w(wg, cin, cout, k):
    w4 = wg.reshape(k, cout, k, cin)
    if k == 5:
        w4 = w4[:, :, _KHG5, :]
    return jnp.transpose(w4, (1, 0, 2, 3)).reshape(cout, k * k * cin)


def kernel(x, conv_w0, conv_b0, conv_w1, conv_b1, conv_w2, conv_b2,
           conv_w3, conv_b3, conv_w4, conv_b4,
           sel0, sel1, sel2, sel3,
           lin_w0, lin_b0, lin_w1, lin_b1, lin_w2, lin_b2, lin_w3, lin_b3):
    B = x.shape[0]
    f32 = jnp.float32
    bf16 = jnp.bfloat16

    xt = jnp.transpose(x, (0, 2, 1, 3))
    xe, xo = xt[:, 0::2], xt[:, 1::2]
    planes = jnp.stack([p.reshape(B, 105, 160)
                        for p in (xe[..., 0::2], xe[..., 1::2],
                                  xo[..., 0::2], xo[..., 1::2])],
                       axis=1).astype(bf16)

    ws = [_reorder_w(w, c[0], c[1], c[2])
          for w, c in zip((conv_w0, conv_w1, conv_w2, conv_w3, conv_w4), _B_CONV)]
    wstk = jnp.zeros((5, 64, 900), bf16)
    bstk = jnp.zeros((5, 64, 1), f32)
    for i, (w, b) in enumerate(zip(ws, (conv_b0, conv_b1, conv_b2, conv_b3, conv_b4))):
        wstk = wstk.at[i, :w.shape[0], :w.shape[1]].set(w.astype(bf16))
        bstk = bstk.at[i, :b.shape[0], :].set(b)
    sstk = jnp.zeros((158, 235), bf16)
    sstk = sstk.at[:, :158].set(jnp.concatenate([sel0, sel1], 1).astype(bf16))
    sstk = sstk.at[:77, 158:].set(jnp.concatenate([sel2, sel3], 1).astype(bf16))

    scratch = [
        pltpu.VMEM((75, 158), bf16), pltpu.VMEM((600, 77), bf16),
        pltpu.VMEM((900, 222), bf16), pltpu.VMEM((432, 140), bf16),
        pltpu.VMEM((576, 66), bf16),
        pltpu.VMEM((408, 79), bf16), pltpu.VMEM((408, 79), bf16),
        pltpu.VMEM((384, 79), bf16), pltpu.VMEM((384, 79), bf16),
        pltpu.VMEM((288, 39), bf16), pltpu.VMEM((288, 38), bf16),
        pltpu.VMEM((252, 39), bf16), pltpu.VMEM((252, 38), bf16),
        pltpu.VMEM((288, 37), bf16), pltpu.VMEM((256, 35), bf16),
    ]
    feat = pl.pallas_call(
        _conv_body,
        out_shape=jax.ShapeDtypeStruct((B, 128, 33), f32),
        grid_spec=pltpu.PrefetchScalarGridSpec(
            num_scalar_prefetch=0,
            grid=(B // _IPS,),
            in_specs=[pl.BlockSpec((_IPS, 4, 105, 160), lambda b: (b, 0, 0, 0)),
                      _cspec(wstk), _cspec(bstk), _cspec(sstk)],
            out_specs=pl.BlockSpec((_IPS, 128, 33), lambda b: (b, 0, 0)),
            scratch_shapes=scratch,
        ),
        compiler_params=pltpu.CompilerParams(dimension_semantics=("parallel",)),
    )(planes, wstk, bstk, sstk)

    xf = feat.reshape(B, 4224)
    lin_consts = [lin_w0, lin_b0, lin_w1, lin_b1, lin_w2, lin_b2, lin_w3, lin_b3]
    mb = B // 2 if B % 2 == 0 else B
    head = pl.pallas_call(
        _head_body,
        out_shape=jax.ShapeDtypeStruct((B, 128), f32),
        grid_spec=pltpu.PrefetchScalarGridSpec(
            num_scalar_prefetch=0,
            grid=(B // mb,),
            in_specs=[pl.BlockSpec((mb, 4224), lambda i: (i, 0))]
                     + [_cspec(a) for a in lin_consts],
            out_specs=pl.BlockSpec((mb, 128), lambda i: (i, 0)),
        ),
        compiler_params=pltpu.CompilerParams(dimension_semantics=("parallel",)),
    )(xf, *lin_consts)
    return head[:, :1]


# speedup vs baseline: 1.7431x; 1.1445x over previous
import jax
import jax.numpy as jnp
from jax.experimental import pallas as pl
from jax.experimental.pallas import tpu as pltpu

_H0, _W0 = 70, 320
_B_CONV = [(3, 24, 5, 2), (24, 36, 5, 2), (36, 48, 5, 2),
           (48, 64, 3, 1), (64, 64, 3, 1)]
_KHG5 = (0, 2, 4, 1, 3)


def _elu(v):
    return jnp.where(v > 0.0, v, jnp.exp(v) - 1.0)


_IPS = 4


def _conv_body(xp, w_ref, b_ref, sel_ref, o_ref,
               slab1, slab2, slab3, slab4, slab5,
               s1ee, s1eo, s1oe, s1oo, s2ee, s2eo, s2oe, s2oo, s3, s4):
    for img in range(_IPS):
        _conv_one(xp, w_ref, b_ref, sel_ref, o_ref, img,
                  slab1, slab2, slab3, slab4, slab5,
                  s1ee, s1eo, s1oe, s1oo, s2ee, s2eo, s2oe, s2oo, s3, s4)


def _conv_one(xp4, w_ref, b_ref, sel_ref, o_ref4, img,
              slab1, slab2, slab3, slab4, slab5,
              s1ee, s1eo, s1oe, s1oo, s2ee, s2eo, s2oe, s2oo, s3, s4):
    f32 = jnp.float32
    bf16 = jnp.bfloat16

    bias1 = jnp.broadcast_to(b_ref[0, :24, :], (24, 158))
    s1 = ((s1ee, s1eo), (s1oe, s1oo))
    w1 = w_ref[0, :24, :75]
    selA = sel_ref[:, :158]
    for r in range(33):
        for kw in range(5):
            dw = kw // 2
            slab1[kw * 15:kw * 15 + 9, :] = xp4[img, kw % 2, 3 * r:3 * r + 9, dw:dw + 158]
            slab1[kw * 15 + 9:kw * 15 + 15, :] = xp4[img, 2 + kw % 2, 3 * r:3 * r + 6, dw:dw + 158]
        acc = _elu(jnp.dot(w1, slab1[...], preferred_element_type=f32) + bias1)
        par = jnp.dot(acc.astype(bf16), selA, preferred_element_type=f32)
        dst = s1[r % 2]
        i = r // 2
        dst[0][24 * i:24 * i + 24, :] = par[:, :79].astype(bf16)
        dst[1][24 * i:24 * i + 24, :] = par[:, 79:].astype(bf16)

    bias2 = jnp.broadcast_to(b_ref[1, :36, :], (36, 77))
    s2 = ((s2ee, s2eo), (s2oe, s2oo))
    w2 = w_ref[1, :36, :600]
    selB = sel_ref[:77, 158:235]
    for r in range(15):
        for kw in range(5):
            dw = kw // 2
            slab2[kw * 120:kw * 120 + 72, :] = s1[0][kw % 2][24 * r:24 * r + 72, dw:dw + 77]
            slab2[kw * 120 + 72:kw * 120 + 120, :] = s1[1][kw % 2][24 * r:24 * r + 48, dw:dw + 77]
        acc = _elu(jnp.dot(w2, slab2[...], preferred_element_type=f32) + bias2)
        par = jnp.dot(acc.astype(bf16), selB, preferred_element_type=f32)
        dst = s2[r % 2]
        i = r // 2
        dst[0][36 * i:36 * i + 36, :] = par[:, :39].astype(bf16)
        dst[1][36 * i:36 * i + 36, :38] = par[:, 39:].astype(bf16)

    for r in range(6):
        for kw in range(5):
            dw = kw // 2
            slab3[kw * 180:kw * 180 + 108, 37 * r:37 * r + 37] = \
                s2[0][kw % 2][36 * r:36 * r + 108, dw:dw + 37]
            slab3[kw * 180 + 108:kw * 180 + 180, 37 * r:37 * r + 37] = \
                s2[1][kw % 2][36 * r:36 * r + 72, dw:dw + 37]
    bias3 = jnp.broadcast_to(b_ref[2, :48, :], (48, 6 * 37))
    out3 = _elu(jnp.dot(w_ref[2, :48, :900], slab3[...],
                        preferred_element_type=f32) + bias3).astype(bf16)
    for r in range(6):
        s3[48 * r:48 * r + 48, :] = out3[:, 37 * r:37 * r + 37]

    for r in range(4):
        for kw in range(3):
            slab4[kw * 144:kw * 144 + 144, 35 * r:35 * r + 35] = \
                s3[48 * r:48 * r + 144, kw:kw + 35]
    bias4 = jnp.broadcast_to(b_ref[3, :64, :], (64, 4 * 35))
    out4 = _elu(jnp.dot(w_ref[3, :64, :432], slab4[...],
                        preferred_element_type=f32) + bias4).astype(bf16)
    for r in range(4):
        s4[64 * r:64 * r + 64, :] = out4[:, 35 * r:35 * r + 35]

    for r in range(2):
        for kw in range(3):
            slab5[kw * 192:kw * 192 + 192, 33 * r:33 * r + 33] = \
                s4[64 * r:64 * r + 192, kw:kw + 33]
    bias5 = jnp.broadcast_to(b_ref[4, :64, :], (64, 2 * 33))
    out5 = jnp.dot(w_ref[4, :64, :576], slab5[...],
                   preferred_element_type=f32) + bias5
    for r in range(2):
        o_ref4[img, 64 * r:64 * r + 64, :] = out5[:, 33 * r:33 * r + 33]


def _head_body(x_ref, w1, b1, w2, b2, w3, b3, w4, b4, o_ref):
    f32 = jnp.float32
    h = x_ref[...]
    h = _elu(jnp.dot(h, w1[...], preferred_element_type=f32) + b1[...])
    h = _elu(jnp.dot(h, w2[...], preferred_element_type=f32) + b2[...])
    h = jnp.dot(h, w3[...], preferred_element_type=f32) + b3[...]
    o_ref[...] = jnp.dot(h, w4[...], preferred_element_type=f32) + b4[...]


def _cspec(a):
    zeros = (0,) * a.ndim
    return pl.BlockSpec(a.shape, lambda b, _z=zeros: _z)


def _reorder_w(wg, cin, cout, k):
    w4 = wg.reshape(k, cout, k, cin)
    if k == 5:
        w4 = w4[:, :, _KHG5, :]
    return jnp.transpose(w4, (1, 0, 2, 3)).reshape(cout, k * k * cin)


def kernel(x, conv_w0, conv_b0, conv_w1, conv_b1, conv_w2, conv_b2,
           conv_w3, conv_b3, conv_w4, conv_b4,
           sel0, sel1, sel2, sel3,
           lin_w0, lin_b0, lin_w1, lin_b1, lin_w2, lin_b2, lin_w3, lin_b3):
    B = x.shape[0]
    f32 = jnp.float32
    bf16 = jnp.bfloat16

    planes = (x.reshape(B, 3, 35, 2, 160, 2)
              .transpose(0, 3, 5, 2, 1, 4)
              .reshape(B, 4, 105, 160).astype(bf16))

    ws = [_reorder_w(w, c[0], c[1], c[2])
          for w, c in zip((conv_w0, conv_w1, conv_w2, conv_w3, conv_w4), _B_CONV)]
    wstk = jnp.zeros((5, 64, 900), bf16)
    bstk = jnp.zeros((5, 64, 1), f32)
    for i, (w, b) in enumerate(zip(ws, (conv_b0, conv_b1, conv_b2, conv_b3, conv_b4))):
        wstk = wstk.at[i, :w.shape[0], :w.shape[1]].set(w.astype(bf16))
        bstk = bstk.at[i, :b.shape[0], :].set(b)
    sstk = jnp.zeros((158, 235), bf16)
    sstk = sstk.at[:, :158].set(jnp.concatenate([sel0, sel1], 1).astype(bf16))
    sstk = sstk.at[:77, 158:].set(jnp.concatenate([sel2, sel3], 1).astype(bf16))

    scratch = [
        pltpu.VMEM((75, 158), bf16), pltpu.VMEM((600, 77), bf16),
        pltpu.VMEM((900, 222), bf16), pltpu.VMEM((432, 140), bf16),
        pltpu.VMEM((576, 66), bf16),
        pltpu.VMEM((408, 79), bf16), pltpu.VMEM((408, 79), bf16),
        pltpu.VMEM((384, 79), bf16), pltpu.VMEM((384, 79), bf16),
        pltpu.VMEM((288, 39), bf16), pltpu.VMEM((288, 38), bf16),
        pltpu.VMEM((252, 39), bf16), pltpu.VMEM((252, 38), bf16),
        pltpu.VMEM((288, 37), bf16), pltpu.VMEM((256, 35), bf16),
    ]
    feat = pl.pallas_call(
        _conv_body,
        out_shape=jax.ShapeDtypeStruct((B, 128, 33), f32),
        grid_spec=pltpu.PrefetchScalarGridSpec(
            num_scalar_prefetch=0,
            grid=(B // _IPS,),
            in_specs=[pl.BlockSpec((_IPS, 4, 105, 160), lambda b: (b, 0, 0, 0)),
                      _cspec(wstk), _cspec(bstk), _cspec(sstk)],
            out_specs=pl.BlockSpec((_IPS, 128, 33), lambda b: (b, 0, 0)),
            scratch_shapes=scratch,
        ),
        compiler_params=pltpu.CompilerParams(dimension_semantics=("parallel",)),
    )(planes, wstk, bstk, sstk)

    xf = feat.reshape(B, 4224)
    lin_consts = [lin_w0, lin_b0, lin_w1, lin_b1, lin_w2, lin_b2, lin_w3, lin_b3]
    mb = B // 2 if B % 2 == 0 else B
    head = pl.pallas_call(
        _head_body,
        out_shape=jax.ShapeDtypeStruct((B, 128), f32),
        grid_spec=pltpu.PrefetchScalarGridSpec(
            num_scalar_prefetch=0,
            grid=(B // mb,),
            in_specs=[pl.BlockSpec((mb, 4224), lambda i: (i, 0))]
                     + [_cspec(a) for a in lin_consts],
            out_specs=pl.BlockSpec((mb, 128), lambda i: (i, 0)),
        ),
        compiler_params=pltpu.CompilerParams(dimension_semantics=("parallel",)),
    )(xf, *lin_consts)
    return head[:, :1]
